# baseline (speedup 1.0000x reference)
import functools

import jax
import jax.numpy as jnp
from jax import lax
from jax.experimental import pallas as pl
from jax.experimental.pallas import tpu as pltpu
from jax.experimental.pallas import tpu_sc as plsc

_NC, _NS, _L = 2, 16, 16
_NW = _NC * _NS
_IDX_CHUNK = 128


def kernel(users, items, user_table, movie_table):
    B = users.shape[0]
    D = user_table.shape[1]
    b_per_w = B // _NW
    n_chunks = b_per_w // _IDX_CHUNK
    mesh = plsc.VectorSubcoreMesh(core_axis_name="c", subcore_axis_name="s")

    @functools.partial(
        pl.kernel,
        out_type=jax.ShapeDtypeStruct((B,), jnp.float32),
        mesh=mesh,
        scratch_types=[
            pltpu.VMEM((b_per_w,), jnp.int32),
            pltpu.VMEM((b_per_w,), jnp.int32),
            pltpu.VMEM((b_per_w, D), jnp.float32),
            pltpu.VMEM((b_per_w, D), jnp.float32),
            pltpu.VMEM((b_per_w,), jnp.float32),
            pltpu.SemaphoreType.DMA,
        ],
        compiler_params=pltpu.CompilerParams(
            needs_layout_passes=False, use_tc_tiling_on_sc=False),
    )
    def mf_kernel(users_hbm, items_hbm, ut_hbm, mt_hbm, out_hbm,
                  uidx_v, iidx_v, urows_v, irows_v, out_v, sem):
        wid = lax.axis_index("s") * _NC + lax.axis_index("c")
        base = wid * b_per_w
        pltpu.sync_copy(users_hbm.at[pl.ds(base, b_per_w)], uidx_v)
        pltpu.sync_copy(items_hbm.at[pl.ds(base, b_per_w)], iidx_v)
        copies = []
        for c in range(n_chunks):
            sl = pl.ds(c * _IDX_CHUNK, _IDX_CHUNK)
            copies.append(
                pltpu.async_copy(ut_hbm.at[uidx_v.at[sl]], urows_v.at[sl], sem))
            copies.append(
                pltpu.async_copy(mt_hbm.at[iidx_v.at[sl]], irows_v.at[sl], sem))
        for cp in copies:
            cp.wait()

        def group_body(g, carry):
            rows = g * _L + lax.iota(jnp.int32, _L)
            acc = jnp.zeros((_L,), jnp.float32)
            for d in range(D):
                dd = jnp.full((_L,), d, jnp.int32)
                u = plsc.load_gather(urows_v, [rows, dd])
                m = plsc.load_gather(irows_v, [rows, dd])
                acc = acc + u * m
            out_v[pl.ds(g * _L, _L)] = 1.0 / (1.0 + jnp.exp(-acc))
            return carry

        lax.fori_loop(0, b_per_w // _L, group_body, 0)
        pltpu.sync_copy(out_v, out_hbm.at[pl.ds(base, b_per_w)])

    return mf_kernel(users, items, user_table, movie_table)
